# baseline (speedup 1.0000x reference)
import functools
import math

import jax
import jax.numpy as jnp
from jax import lax
from jax.experimental import pallas as pl
from jax.experimental.pallas import tpu as pltpu
from jax.experimental.pallas import tpu_sc as plsc

H = 128
NUM_RBF = 128
N_NODES = 10000
N_EDGES = 320000

_INV3 = 1.0 / math.sqrt(3.0)
_INVH = 1.0 / math.sqrt(H)

_K0_BLK = 1000


def _k0_body(x_ref, vec_ref, w1_ref, b1_ref, w2_ref, b2_ref, t_ref):
    h = jnp.dot(x_ref[...], w1_ref[...], preferred_element_type=jnp.float32)
    h = h + b1_ref[...]
    h = jax.nn.silu(h) * (1.0 / 0.6)
    xh = jnp.dot(h, w2_ref[...], preferred_element_type=jnp.float32)
    xh = xh + b2_ref[...]
    t_ref[:, 0:384] = xh
    t_ref[:, 384:768] = vec_ref[...]


def _build_table(x, vec_flat, W1, b1, W2, b2):
    nblk = N_NODES // _K0_BLK
    return pl.pallas_call(
        _k0_body,
        grid=(nblk,),
        in_specs=[
            pl.BlockSpec((_K0_BLK, H), lambda i: (i, 0)),
            pl.BlockSpec((_K0_BLK, 384), lambda i: (i, 0)),
            pl.BlockSpec((H, H // 2), lambda i: (0, 0)),
            pl.BlockSpec((1, H // 2), lambda i: (0, 0)),
            pl.BlockSpec((H // 2, 384), lambda i: (0, 0)),
            pl.BlockSpec((1, 384), lambda i: (0, 0)),
        ],
        out_specs=pl.BlockSpec((_K0_BLK, 768), lambda i: (i, 0)),
        out_shape=jax.ShapeDtypeStruct((N_NODES, 768), jnp.float32),
    )(x, vec_flat, W1, b1, W2, b2)


_NC = 2
_NS = 16
_NW = _NC * _NS
_GCH = 80
_GSTRIPE = N_EDGES // _NW
_GCHUNKS = _GSTRIPE // _GCH


def _ka_body(t_hbm, j_hbm, ej_hbm, jbuf0, jbuf1, rbuf0, rbuf1, sem0, sem1):
    wid = lax.axis_index("s") * _NC + lax.axis_index("c")
    stripe0 = wid * _GSTRIPE

    def start(g, jbuf, rbuf, sem):
        base = stripe0 + g * _GCH
        pltpu.sync_copy(j_hbm.at[pl.ds(base, _GCH)], jbuf)
        return pltpu.async_copy(t_hbm.at[jbuf], rbuf, sem)

    def store(g, rbuf):
        base = stripe0 + g * _GCH
        pltpu.sync_copy(rbuf, ej_hbm.at[pl.ds(base, _GCH)])

    def drain(rbuf, sem):
        pltpu.make_async_copy(t_hbm.at[jbuf0], rbuf, sem).wait()

    start(0, jbuf0, rbuf0, sem0)

    def pair(h, carry):
        g0 = 2 * h
        start(g0 + 1, jbuf1, rbuf1, sem1)
        drain(rbuf0, sem0)
        store(g0, rbuf0)
        start(g0 + 2, jbuf0, rbuf0, sem0)
        drain(rbuf1, sem1)
        store(g0 + 1, rbuf1)
        return carry

    lax.fori_loop(0, (_GCHUNKS - 1) // 2, pair, 0)
    drain(rbuf0, sem0)
    store(_GCHUNKS - 1, rbuf0)


def _gather_edges(table, j_idx):
    mesh = plsc.VectorSubcoreMesh(core_axis_name="c", subcore_axis_name="s")
    return pl.kernel(
        _ka_body,
        out_type=jax.ShapeDtypeStruct((N_EDGES, 768), jnp.float32),
        mesh=mesh,
        compiler_params=pltpu.CompilerParams(needs_layout_passes=False),
        scratch_types=[
            pltpu.VMEM((_GCH,), jnp.int32),
            pltpu.VMEM((_GCH,), jnp.int32),
            pltpu.VMEM((_GCH, 768), jnp.float32),
            pltpu.VMEM((_GCH, 768), jnp.float32),
            pltpu.SemaphoreType.DMA,
            pltpu.SemaphoreType.DMA,
        ],
    )(table, j_idx)


_KB_BLK = 1000


def _kb_body(rbf_ref, ej_ref, u0_ref, u1_ref, u2_ref, we_ref, be_ref, out_ref):
    rbf_h = jnp.dot(rbf_ref[...], we_ref[...], preferred_element_type=jnp.float32)
    rbf_h = rbf_h + be_ref[...]
    msg = ej_ref[:, 0:384] * rbf_h * _INV3
    x1 = msg[:, 0:128]
    x2 = msg[:, 128:256]
    us = (u0_ref[...], u1_ref[...], u2_ref[...])
    for d in range(3):
        vj = ej_ref[:, 384 + d * 128:384 + (d + 1) * 128]
        out_ref[:, d * 128:(d + 1) * 128] = (x1 * vj + x2 * us[d]) * _INVH
    out_ref[:, 384:512] = msg[:, 256:384]


def _edge_dense(edge_rbf, ej, u0, u1, u2, We, be):
    nblk = N_EDGES // _KB_BLK
    return pl.pallas_call(
        _kb_body,
        grid=(nblk,),
        in_specs=[
            pl.BlockSpec((_KB_BLK, NUM_RBF), lambda i: (i, 0)),
            pl.BlockSpec((_KB_BLK, 768), lambda i: (i, 0)),
            pl.BlockSpec((_KB_BLK, 1), lambda i: (i, 0)),
            pl.BlockSpec((_KB_BLK, 1), lambda i: (i, 0)),
            pl.BlockSpec((_KB_BLK, 1), lambda i: (i, 0)),
            pl.BlockSpec((NUM_RBF, 384), lambda i: (0, 0)),
            pl.BlockSpec((1, 384), lambda i: (0, 0)),
        ],
        out_specs=pl.BlockSpec((_KB_BLK, 512), lambda i: (i, 0)),
        out_shape=jax.ShapeDtypeStruct((N_EDGES, 512), jnp.float32),
    )(edge_rbf, ej, u0, u1, u2, We, be)


_P = 2
_RANGE = 160
_NPAD = _P * _NW * _RANGE
_SLAB = 168
_SCCH = 4000
_SVREG = _SCCH // 16
_SCHUNKS = N_EDGES // _SCCH
_G = 48


_LCAP = 4160
_GDN = lax.GatherDimensionNumbers(offset_dims=(), collapsed_slice_dims=(0,),
                                  start_index_map=(0,))


def _lane_bcast(vec, g):
    idx = jnp.full((16,), g, jnp.int32)
    return lax.gather(vec, idx[:, None], _GDN, (1,),
                      mode=lax.GatherScatterMode.PROMISE_IN_BOUNDS)


def _kc_body(rows_hbm, i_hbm, zeros_hbm, outf_hbm,
             scanbuf, eidbuf, locbuf, rstage, slabf, sem):
    c = lax.axis_index("c")
    s = lax.axis_index("s")
    w = s * _NC + c

    iota16 = lax.iota(jnp.int32, 16)
    dummy_eid = jnp.zeros((16,), jnp.int32)
    dummy_loc = jnp.full((16,), _RANGE, jnp.int32)

    def accumulate_batch(off):
        pltpu.async_copy(rows_hbm.at[eidbuf.at[pl.ds(off, _G)]], rstage,
                         sem).wait()
        for vv in range(_G // 16):
            locv = locbuf[pl.ds(off + vv * 16, 16)]

            def acc_row(l, carry):
                base = _lane_bcast(locv, l) * 512 + iota16
                row = vv * 16 + l
                data = rstage[row, pl.ds(0, 16)]
                for k in range(31):
                    nxt = rstage[row, pl.ds((k + 1) * 16, 16)]
                    plsc.addupdate_scatter(slabf, [base + k * 16], data)
                    data = nxt
                plsc.addupdate_scatter(slabf, [base + 31 * 16], data)
                return carry

            lax.fori_loop(0, 16, acc_row, 0)

    for p in range(_P):
        rid = p * _NW + w
        lo = rid * _RANGE
        hi = lo + _RANGE
        pltpu.sync_copy(zeros_hbm, slabf)

        def scan_chunk(ch, cntv):
            base = ch * _SCCH
            pltpu.sync_copy(i_hbm.at[pl.ds(base, _SCCH)], scanbuf)

            def vloop(k2, cntv):
                k = k2 * 2
                va = scanbuf[pl.ds(k * 16, 16)]
                vb = scanbuf[pl.ds(k * 16 + 16, 16)]
                ma = (va >= lo) & (va < hi)
                mb = (vb >= lo) & (vb < hi)
                mia = ma.astype(jnp.int32)
                mib = mb.astype(jnp.int32)
                incla = plsc.cumsum(mia)
                inclb = plsc.cumsum(mib)
                pca = plsc.all_reduce_population_count(ma)
                pcb = plsc.all_reduce_population_count(mb)
                posa = cntv + incla - mia
                cntb = cntv + pca
                posb = cntb + inclb - mib
                eida = base + k * 16 + iota16
                plsc.store_scatter(eidbuf, [posa], eida, mask=ma)
                plsc.store_scatter(locbuf, [posa], va - lo, mask=ma)
                plsc.store_scatter(eidbuf, [posb], eida + 16, mask=mb)
                plsc.store_scatter(locbuf, [posb], vb - lo, mask=mb)
                return cntb + pcb

            cntv = lax.fori_loop(0, _SVREG // 2, vloop, cntv)
            cnt = cntv[0]
            nb = cnt // _G

            def bloop(b, carry):
                accumulate_batch(b * _G)
                return carry

            lax.fori_loop(0, nb, bloop, 0)
            tail_off = nb * _G
            for t in range(3):
                tv = eidbuf[pl.ds(tail_off + t * 16, 16)]
                eidbuf[pl.ds(t * 16, 16)] = tv
                lv = locbuf[pl.ds(tail_off + t * 16, 16)]
                locbuf[pl.ds(t * 16, 16)] = lv
            return cntv - nb * _G

        cntv = lax.fori_loop(0, _SCHUNKS, scan_chunk,
                             jnp.zeros((16,), jnp.int32))

        cnt = cntv[0]
        for t in range(3):
            plsc.store_scatter(eidbuf, [cnt + t * 16 + iota16], dummy_eid)
            plsc.store_scatter(locbuf, [cnt + t * 16 + iota16], dummy_loc)
        accumulate_batch(0)

        pltpu.sync_copy(slabf.at[pl.ds(0, _RANGE * 512)],
                        outf_hbm.at[pl.ds(lo * 512, _RANGE * 512)])


def _scatter_rows(rows, i_idx, zeros):
    mesh = plsc.VectorSubcoreMesh(core_axis_name="c", subcore_axis_name="s")
    return pl.kernel(
        _kc_body,
        out_type=jax.ShapeDtypeStruct((_NPAD * 512,), jnp.float32),
        mesh=mesh,
        compiler_params=pltpu.CompilerParams(needs_layout_passes=False),
        scratch_types=[
            pltpu.VMEM((_SCCH,), jnp.int32),
            pltpu.VMEM((_LCAP,), jnp.int32),
            pltpu.VMEM((_LCAP,), jnp.int32),
            pltpu.VMEM((_G, 512), jnp.float32),
            pltpu.VMEM((_SLAB * 512,), jnp.float32),
            pltpu.SemaphoreType.DMA,
        ],
    )(rows, i_idx, zeros)


def kernel(x, vec, edge_index, edge_rbf, edge_udiff, W1, b1, W2, b2, We, be):
    j = edge_index[0].astype(jnp.int32)
    i = edge_index[1].astype(jnp.int32)
    vec_flat = vec.reshape(N_NODES, 384)
    table = _build_table(x, vec_flat, W1, b1.reshape(1, -1), W2,
                         b2.reshape(1, -1))
    ej = _gather_edges(table, j)
    u0 = edge_udiff[:, 0:1]
    u1 = edge_udiff[:, 1:2]
    u2 = edge_udiff[:, 2:3]
    rows = _edge_dense(edge_rbf, ej, u0, u1, u2, We, be.reshape(1, -1))
    zeros = jnp.zeros((_SLAB * 512,), jnp.float32)
    out = _scatter_rows(rows, i, zeros).reshape(_NPAD, 512)[:N_NODES]
    d_vec = out[:, 0:384].reshape(N_NODES, 3, H)
    d_x = out[:, 384:512]
    return (d_x, d_vec)


# speedup vs baseline: 1.0636x; 1.0636x over previous
import functools
import math

import jax
import jax.numpy as jnp
from jax import lax
from jax.experimental import pallas as pl
from jax.experimental.pallas import tpu as pltpu
from jax.experimental.pallas import tpu_sc as plsc

H = 128
NUM_RBF = 128
N_NODES = 10000
N_EDGES = 320000

_INV3 = 1.0 / math.sqrt(3.0)
_INVH = 1.0 / math.sqrt(H)

_K0_BLK = 1000


def _k0_body(x_ref, vec_ref, w1_ref, b1_ref, w2_ref, b2_ref, t_ref):
    h = jnp.dot(x_ref[...], w1_ref[...], preferred_element_type=jnp.float32)
    h = h + b1_ref[...]
    h = jax.nn.silu(h) * (1.0 / 0.6)
    xh = jnp.dot(h, w2_ref[...], preferred_element_type=jnp.float32)
    xh = xh + b2_ref[...]
    t_ref[:, 0:384] = xh
    t_ref[:, 384:768] = vec_ref[...]


def _build_table(x, vec_flat, W1, b1, W2, b2):
    nblk = N_NODES // _K0_BLK
    return pl.pallas_call(
        _k0_body,
        grid=(nblk,),
        in_specs=[
            pl.BlockSpec((_K0_BLK, H), lambda i: (i, 0)),
            pl.BlockSpec((_K0_BLK, 384), lambda i: (i, 0)),
            pl.BlockSpec((H, H // 2), lambda i: (0, 0)),
            pl.BlockSpec((1, H // 2), lambda i: (0, 0)),
            pl.BlockSpec((H // 2, 384), lambda i: (0, 0)),
            pl.BlockSpec((1, 384), lambda i: (0, 0)),
        ],
        out_specs=pl.BlockSpec((_K0_BLK, 768), lambda i: (i, 0)),
        out_shape=jax.ShapeDtypeStruct((N_NODES, 768), jnp.float32),
    )(x, vec_flat, W1, b1, W2, b2)


_NC = 2
_NS = 16
_NW = _NC * _NS
_GCH = 80
_GSTRIPE = N_EDGES // _NW
_GCHUNKS = _GSTRIPE // _GCH


def _ka_body(t_hbm, j_hbm, ej_hbm, jbuf0, jbuf1, rbuf0, rbuf1, sem0, sem1):
    wid = lax.axis_index("s") * _NC + lax.axis_index("c")
    stripe0 = wid * _GSTRIPE

    def start(g, jbuf, rbuf, sem):
        base = stripe0 + g * _GCH
        pltpu.sync_copy(j_hbm.at[pl.ds(base, _GCH)], jbuf)
        return pltpu.async_copy(t_hbm.at[jbuf], rbuf, sem)

    def store(g, rbuf):
        base = stripe0 + g * _GCH
        pltpu.sync_copy(rbuf, ej_hbm.at[pl.ds(base, _GCH)])

    def drain(rbuf, sem):
        pltpu.make_async_copy(t_hbm.at[jbuf0], rbuf, sem).wait()

    start(0, jbuf0, rbuf0, sem0)

    def pair(h, carry):
        g0 = 2 * h
        start(g0 + 1, jbuf1, rbuf1, sem1)
        drain(rbuf0, sem0)
        store(g0, rbuf0)
        start(g0 + 2, jbuf0, rbuf0, sem0)
        drain(rbuf1, sem1)
        store(g0 + 1, rbuf1)
        return carry

    lax.fori_loop(0, (_GCHUNKS - 1) // 2, pair, 0)
    drain(rbuf0, sem0)
    store(_GCHUNKS - 1, rbuf0)


def _gather_edges(table, j_idx):
    mesh = plsc.VectorSubcoreMesh(core_axis_name="c", subcore_axis_name="s")
    return pl.kernel(
        _ka_body,
        out_type=jax.ShapeDtypeStruct((N_EDGES, 768), jnp.float32),
        mesh=mesh,
        compiler_params=pltpu.CompilerParams(needs_layout_passes=False),
        scratch_types=[
            pltpu.VMEM((_GCH,), jnp.int32),
            pltpu.VMEM((_GCH,), jnp.int32),
            pltpu.VMEM((_GCH, 768), jnp.float32),
            pltpu.VMEM((_GCH, 768), jnp.float32),
            pltpu.SemaphoreType.DMA,
            pltpu.SemaphoreType.DMA,
        ],
    )(table, j_idx)


_KB_BLK = 1000


def _kb_body(rbf_ref, ej_ref, u0_ref, u1_ref, u2_ref, we_ref, be_ref, out_ref):
    rbf_h = jnp.dot(rbf_ref[...], we_ref[...], preferred_element_type=jnp.float32)
    rbf_h = rbf_h + be_ref[...]
    msg = ej_ref[:, 0:384] * rbf_h * _INV3
    x1 = msg[:, 0:128]
    x2 = msg[:, 128:256]
    us = (u0_ref[...], u1_ref[...], u2_ref[...])
    for d in range(3):
        vj = ej_ref[:, 384 + d * 128:384 + (d + 1) * 128]
        out_ref[:, d * 128:(d + 1) * 128] = (x1 * vj + x2 * us[d]) * _INVH
    out_ref[:, 384:512] = msg[:, 256:384]


def _edge_dense(edge_rbf, ej, u0, u1, u2, We, be):
    nblk = N_EDGES // _KB_BLK
    return pl.pallas_call(
        _kb_body,
        grid=(nblk,),
        in_specs=[
            pl.BlockSpec((_KB_BLK, NUM_RBF), lambda i: (i, 0)),
            pl.BlockSpec((_KB_BLK, 768), lambda i: (i, 0)),
            pl.BlockSpec((_KB_BLK, 1), lambda i: (i, 0)),
            pl.BlockSpec((_KB_BLK, 1), lambda i: (i, 0)),
            pl.BlockSpec((_KB_BLK, 1), lambda i: (i, 0)),
            pl.BlockSpec((NUM_RBF, 384), lambda i: (0, 0)),
            pl.BlockSpec((1, 384), lambda i: (0, 0)),
        ],
        out_specs=pl.BlockSpec((_KB_BLK, 512), lambda i: (i, 0)),
        out_shape=jax.ShapeDtypeStruct((N_EDGES, 512), jnp.float32),
    )(edge_rbf, ej, u0, u1, u2, We, be)


_P = 2
_RANGE = 160
_NPAD = _P * _NW * _RANGE
_SLAB = 168
_SCCH = 3200
_SVREG = _SCCH // 16
_SCHUNKS = N_EDGES // _SCCH
_G = 32


_LCAP = 3264
_GDN = lax.GatherDimensionNumbers(offset_dims=(), collapsed_slice_dims=(0,),
                                  start_index_map=(0,))


def _lane_bcast(vec, g):
    idx = jnp.full((16,), g, jnp.int32)
    return lax.gather(vec, idx[:, None], _GDN, (1,),
                      mode=lax.GatherScatterMode.PROMISE_IN_BOUNDS)


def _kc_body(rows_hbm, i_hbm, zeros_hbm, outf_hbm,
             scanbuf, eidbuf, locbuf, rstage0, rstage1, slabf, sem0, sem1):
    c = lax.axis_index("c")
    s = lax.axis_index("s")
    w = s * _NC + c

    iota16 = lax.iota(jnp.int32, 16)
    dummy_eid = jnp.zeros((16,), jnp.int32)
    dummy_loc = jnp.full((16,), _RANGE, jnp.int32)

    def gstart(off, rstage, sem):
        pltpu.async_copy(rows_hbm.at[eidbuf.at[pl.ds(off, _G)]], rstage, sem)

    def gdrain(rstage, sem):
        pltpu.make_async_copy(rows_hbm.at[eidbuf.at[pl.ds(0, _G)]], rstage,
                              sem).wait()

    def accum(off, rstage):
        for vv in range(_G // 16):
            locv = locbuf[pl.ds(off + vv * 16, 16)]

            def acc_row(l, carry):
                base = _lane_bcast(locv, l) * 512 + iota16
                row = vv * 16 + l
                d0 = rstage[row, pl.ds(0, 16)]
                d1 = rstage[row, pl.ds(16, 16)]
                for k in range(30):
                    nxt = rstage[row, pl.ds((k + 2) * 16, 16)]
                    plsc.addupdate_scatter(slabf, [base + k * 16], d0)
                    d0 = d1
                    d1 = nxt
                plsc.addupdate_scatter(slabf, [base + 30 * 16], d0)
                plsc.addupdate_scatter(slabf, [base + 31 * 16], d1)
                return carry

            lax.fori_loop(0, 16, acc_row, 0)

    for p in range(_P):
        rid = p * _NW + w
        lo = rid * _RANGE
        hi = lo + _RANGE
        pltpu.sync_copy(zeros_hbm, slabf)

        def scan_chunk(ch, cntv):
            base = ch * _SCCH
            pltpu.sync_copy(i_hbm.at[pl.ds(base, _SCCH)], scanbuf)

            def vloop(k2, cntv):
                k = k2 * 2
                va = scanbuf[pl.ds(k * 16, 16)]
                vb = scanbuf[pl.ds(k * 16 + 16, 16)]
                ma = (va >= lo) & (va < hi)
                mb = (vb >= lo) & (vb < hi)
                mia = ma.astype(jnp.int32)
                mib = mb.astype(jnp.int32)
                incla = plsc.cumsum(mia)
                inclb = plsc.cumsum(mib)
                pca = plsc.all_reduce_population_count(ma)
                pcb = plsc.all_reduce_population_count(mb)
                posa = cntv + incla - mia
                cntb = cntv + pca
                posb = cntb + inclb - mib
                eida = base + k * 16 + iota16
                plsc.store_scatter(eidbuf, [posa], eida, mask=ma)
                plsc.store_scatter(locbuf, [posa], va - lo, mask=ma)
                plsc.store_scatter(eidbuf, [posb], eida + 16, mask=mb)
                plsc.store_scatter(locbuf, [posb], vb - lo, mask=mb)
                return cntb + pcb

            cntv = lax.fori_loop(0, _SVREG // 2, vloop, cntv)
            cnt = cntv[0]
            nb = cnt // _G

            @pl.when(nb > 0)
            def _():
                gstart(0, rstage0, sem0)

            def bpair(h, carry):
                b0 = 2 * h

                @pl.when(b0 + 1 < nb)
                def _():
                    gstart((b0 + 1) * _G, rstage1, sem1)

                gdrain(rstage0, sem0)
                accum(b0 * _G, rstage0)

                @pl.when(b0 + 2 < nb)
                def _():
                    gstart((b0 + 2) * _G, rstage0, sem0)

                @pl.when(b0 + 1 < nb)
                def _():
                    gdrain(rstage1, sem1)
                    accum((b0 + 1) * _G, rstage1)

                return carry

            lax.fori_loop(0, (nb + 1) // 2, bpair, 0)
            tail_off = nb * _G
            for t in range(2):
                tv = eidbuf[pl.ds(tail_off + t * 16, 16)]
                eidbuf[pl.ds(t * 16, 16)] = tv
                lv = locbuf[pl.ds(tail_off + t * 16, 16)]
                locbuf[pl.ds(t * 16, 16)] = lv
            return cntv - nb * _G

        cntv = lax.fori_loop(0, _SCHUNKS, scan_chunk,
                             jnp.zeros((16,), jnp.int32))

        cnt = cntv[0]
        for t in range(2):
            plsc.store_scatter(eidbuf, [cnt + t * 16 + iota16], dummy_eid)
            plsc.store_scatter(locbuf, [cnt + t * 16 + iota16], dummy_loc)
        gstart(0, rstage0, sem0)
        gdrain(rstage0, sem0)
        accum(0, rstage0)

        pltpu.sync_copy(slabf.at[pl.ds(0, _RANGE * 512)],
                        outf_hbm.at[pl.ds(lo * 512, _RANGE * 512)])


def _scatter_rows(rows, i_idx, zeros):
    mesh = plsc.VectorSubcoreMesh(core_axis_name="c", subcore_axis_name="s")
    return pl.kernel(
        _kc_body,
        out_type=jax.ShapeDtypeStruct((_NPAD * 512,), jnp.float32),
        mesh=mesh,
        compiler_params=pltpu.CompilerParams(needs_layout_passes=False),
        scratch_types=[
            pltpu.VMEM((_SCCH,), jnp.int32),
            pltpu.VMEM((_LCAP,), jnp.int32),
            pltpu.VMEM((_LCAP,), jnp.int32),
            pltpu.VMEM((_G, 512), jnp.float32),
            pltpu.VMEM((_G, 512), jnp.float32),
            pltpu.VMEM((_SLAB * 512,), jnp.float32),
            pltpu.SemaphoreType.DMA,
            pltpu.SemaphoreType.DMA,
        ],
    )(rows, i_idx, zeros)


def kernel(x, vec, edge_index, edge_rbf, edge_udiff, W1, b1, W2, b2, We, be):
    j = edge_index[0].astype(jnp.int32)
    i = edge_index[1].astype(jnp.int32)
    vec_flat = vec.reshape(N_NODES, 384)
    table = _build_table(x, vec_flat, W1, b1.reshape(1, -1), W2,
                         b2.reshape(1, -1))
    ej = _gather_edges(table, j)
    u0 = edge_udiff[:, 0:1]
    u1 = edge_udiff[:, 1:2]
    u2 = edge_udiff[:, 2:3]
    rows = _edge_dense(edge_rbf, ej, u0, u1, u2, We, be.reshape(1, -1))
    zeros = jnp.zeros((_SLAB * 512,), jnp.float32)
    out = _scatter_rows(rows, i, zeros).reshape(_NPAD, 512)[:N_NODES]
    d_vec = out[:, 0:384].reshape(N_NODES, 3, H)
    d_x = out[:, 384:512]
    return (d_x, d_vec)


# speedup vs baseline: 1.1070x; 1.0407x over previous
import functools
import math

import jax
import jax.numpy as jnp
from jax import lax
from jax.experimental import pallas as pl
from jax.experimental.pallas import tpu as pltpu
from jax.experimental.pallas import tpu_sc as plsc

H = 128
NUM_RBF = 128
N_NODES = 10000
N_EDGES = 320000

_INV3 = 1.0 / math.sqrt(3.0)
_INVH = 1.0 / math.sqrt(H)

_K0_BLK = 1000


def _k0_body(x_ref, vec_ref, w1_ref, b1_ref, w2_ref, b2_ref, t_ref):
    h = jnp.dot(x_ref[...], w1_ref[...], preferred_element_type=jnp.float32)
    h = h + b1_ref[...]
    h = jax.nn.silu(h) * (1.0 / 0.6)
    xh = jnp.dot(h, w2_ref[...], preferred_element_type=jnp.float32)
    xh = xh + b2_ref[...]
    t_ref[:, 0:384] = xh
    t_ref[:, 384:768] = vec_ref[...]


def _build_table(x, vec_flat, W1, b1, W2, b2):
    nblk = N_NODES // _K0_BLK
    return pl.pallas_call(
        _k0_body,
        grid=(nblk,),
        in_specs=[
            pl.BlockSpec((_K0_BLK, H), lambda i: (i, 0)),
            pl.BlockSpec((_K0_BLK, 384), lambda i: (i, 0)),
            pl.BlockSpec((H, H // 2), lambda i: (0, 0)),
            pl.BlockSpec((1, H // 2), lambda i: (0, 0)),
            pl.BlockSpec((H // 2, 384), lambda i: (0, 0)),
            pl.BlockSpec((1, 384), lambda i: (0, 0)),
        ],
        out_specs=pl.BlockSpec((_K0_BLK, 768), lambda i: (i, 0)),
        out_shape=jax.ShapeDtypeStruct((N_NODES, 768), jnp.float32),
    )(x, vec_flat, W1, b1, W2, b2)


_NC = 2
_NS = 16
_NW = _NC * _NS
_GCH = 80
_GSTRIPE = N_EDGES // _NW
_GCHUNKS = _GSTRIPE // _GCH


def _ka_body(t_hbm, j_hbm, ej_hbm, jbuf0, jbuf1, rbuf0, rbuf1, sem0, sem1):
    wid = lax.axis_index("s") * _NC + lax.axis_index("c")
    stripe0 = wid * _GSTRIPE

    def start(g, jbuf, rbuf, sem):
        base = stripe0 + g * _GCH
        pltpu.sync_copy(j_hbm.at[pl.ds(base, _GCH)], jbuf)
        return pltpu.async_copy(t_hbm.at[jbuf], rbuf, sem)

    def store(g, rbuf):
        base = stripe0 + g * _GCH
        pltpu.sync_copy(rbuf, ej_hbm.at[pl.ds(base, _GCH)])

    def drain(rbuf, sem):
        pltpu.make_async_copy(t_hbm.at[jbuf0], rbuf, sem).wait()

    start(0, jbuf0, rbuf0, sem0)

    def pair(h, carry):
        g0 = 2 * h
        start(g0 + 1, jbuf1, rbuf1, sem1)
        drain(rbuf0, sem0)
        store(g0, rbuf0)
        start(g0 + 2, jbuf0, rbuf0, sem0)
        drain(rbuf1, sem1)
        store(g0 + 1, rbuf1)
        return carry

    lax.fori_loop(0, (_GCHUNKS - 1) // 2, pair, 0)
    drain(rbuf0, sem0)
    store(_GCHUNKS - 1, rbuf0)


def _gather_edges(table, j_idx):
    mesh = plsc.VectorSubcoreMesh(core_axis_name="c", subcore_axis_name="s")
    return pl.kernel(
        _ka_body,
        out_type=jax.ShapeDtypeStruct((N_EDGES, 768), jnp.float32),
        mesh=mesh,
        compiler_params=pltpu.CompilerParams(needs_layout_passes=False),
        scratch_types=[
            pltpu.VMEM((_GCH,), jnp.int32),
            pltpu.VMEM((_GCH,), jnp.int32),
            pltpu.VMEM((_GCH, 768), jnp.float32),
            pltpu.VMEM((_GCH, 768), jnp.float32),
            pltpu.SemaphoreType.DMA,
            pltpu.SemaphoreType.DMA,
        ],
    )(table, j_idx)


_KB_BLK = 1000


def _kb_body(rbf_ref, ej_ref, u0_ref, u1_ref, u2_ref, we_ref, be_ref, out_ref):
    rbf_h = jnp.dot(rbf_ref[...], we_ref[...], preferred_element_type=jnp.float32)
    rbf_h = rbf_h + be_ref[...]
    msg = ej_ref[:, 0:384] * rbf_h * _INV3
    x1 = msg[:, 0:128]
    x2 = msg[:, 128:256]
    us = (u0_ref[...], u1_ref[...], u2_ref[...])
    for d in range(3):
        vj = ej_ref[:, 384 + d * 128:384 + (d + 1) * 128]
        out_ref[:, d * 128:(d + 1) * 128] = (x1 * vj + x2 * us[d]) * _INVH
    out_ref[:, 384:512] = msg[:, 256:384]


def _edge_dense(edge_rbf, ej, u0, u1, u2, We, be):
    nblk = N_EDGES // _KB_BLK
    return pl.pallas_call(
        _kb_body,
        grid=(nblk,),
        in_specs=[
            pl.BlockSpec((_KB_BLK, NUM_RBF), lambda i: (i, 0)),
            pl.BlockSpec((_KB_BLK, 768), lambda i: (i, 0)),
            pl.BlockSpec((_KB_BLK, 1), lambda i: (i, 0)),
            pl.BlockSpec((_KB_BLK, 1), lambda i: (i, 0)),
            pl.BlockSpec((_KB_BLK, 1), lambda i: (i, 0)),
            pl.BlockSpec((NUM_RBF, 384), lambda i: (0, 0)),
            pl.BlockSpec((1, 384), lambda i: (0, 0)),
        ],
        out_specs=pl.BlockSpec((_KB_BLK, 512), lambda i: (i, 0)),
        out_shape=jax.ShapeDtypeStruct((N_EDGES, 512), jnp.float32),
    )(edge_rbf, ej, u0, u1, u2, We, be)


_P = 2
_RANGE = 160
_NPAD = _P * _NW * _RANGE
_SLAB = 168
_SCCH = 3200
_SVREG = _SCCH // 16
_SCHUNKS = N_EDGES // _SCCH
_G = 32


_LCAP = 3264
_GDN = lax.GatherDimensionNumbers(offset_dims=(), collapsed_slice_dims=(0,),
                                  start_index_map=(0,))


def _lane_bcast(vec, g):
    idx = jnp.full((16,), g, jnp.int32)
    return lax.gather(vec, idx[:, None], _GDN, (1,),
                      mode=lax.GatherScatterMode.PROMISE_IN_BOUNDS)


def _kc_body(rows_hbm, i_hbm, zeros_hbm, outf_hbm,
             scanbuf, eidbuf, locbuf, rstage0, rstage1, slabf, sem0, sem1):
    c = lax.axis_index("c")
    s = lax.axis_index("s")
    w = s * _NC + c

    iota16 = lax.iota(jnp.int32, 16)
    dummy_eid = jnp.zeros((16,), jnp.int32)
    dummy_loc = jnp.full((16,), _RANGE, jnp.int32)

    def gstart(off, rstage, sem):
        pltpu.async_copy(rows_hbm.at[eidbuf.at[pl.ds(off, _G)]], rstage, sem)

    def gdrain(rstage, sem):
        pltpu.make_async_copy(rows_hbm.at[eidbuf.at[pl.ds(0, _G)]], rstage,
                              sem).wait()

    def accum(off, rstage):
        for vv in range(_G // 16):
            locv = locbuf[pl.ds(off + vv * 16, 16)]

            def acc_row(l, carry):
                base = _lane_bcast(locv, l) * 512 + iota16
                row = vv * 16 + l
                d0 = rstage[row, pl.ds(0, 16)]
                d1 = rstage[row, pl.ds(16, 16)]
                for k in range(30):
                    nxt = rstage[row, pl.ds((k + 2) * 16, 16)]
                    plsc.addupdate_scatter(slabf, [base + k * 16], d0)
                    d0 = d1
                    d1 = nxt
                plsc.addupdate_scatter(slabf, [base + 30 * 16], d0)
                plsc.addupdate_scatter(slabf, [base + 31 * 16], d1)
                return carry

            lax.fori_loop(0, 16, acc_row, 0)

    for p in range(_P):
        rid = p * _NW + w
        lo = rid * _RANGE
        hi = lo + _RANGE
        pltpu.sync_copy(zeros_hbm, slabf)

        def scan_chunk(ch, cntv):
            base = ch * _SCCH
            pltpu.sync_copy(i_hbm.at[pl.ds(base, _SCCH)], scanbuf)

            def vloop(k4, cntv):
                k = k4 * 4
                vs = [scanbuf[pl.ds((k + q) * 16, 16)] for q in range(4)]
                ms = [(v >= lo) & (v < hi) for v in vs]
                mis = [m.astype(jnp.int32) for m in ms]
                incls = [plsc.cumsum(mi) for mi in mis]
                pcs = [plsc.all_reduce_population_count(m) for m in ms]
                for q in range(4):
                    pos = cntv + incls[q] - mis[q]
                    eid = base + (k + q) * 16 + iota16
                    plsc.store_scatter(eidbuf, [pos], eid, mask=ms[q])
                    plsc.store_scatter(locbuf, [pos], vs[q] - lo, mask=ms[q])
                    cntv = cntv + pcs[q]
                return cntv

            cntv = lax.fori_loop(0, _SVREG // 4, vloop, cntv)
            cnt = cntv[0]
            nb = cnt // _G

            @pl.when(nb > 0)
            def _():
                gstart(0, rstage0, sem0)

            def bpair(h, carry):
                b0 = 2 * h

                @pl.when(b0 + 1 < nb)
                def _():
                    gstart((b0 + 1) * _G, rstage1, sem1)

                gdrain(rstage0, sem0)
                accum(b0 * _G, rstage0)

                @pl.when(b0 + 2 < nb)
                def _():
                    gstart((b0 + 2) * _G, rstage0, sem0)

                @pl.when(b0 + 1 < nb)
                def _():
                    gdrain(rstage1, sem1)
                    accum((b0 + 1) * _G, rstage1)

                return carry

            lax.fori_loop(0, (nb + 1) // 2, bpair, 0)
            tail_off = nb * _G
            for t in range(2):
                tv = eidbuf[pl.ds(tail_off + t * 16, 16)]
                eidbuf[pl.ds(t * 16, 16)] = tv
                lv = locbuf[pl.ds(tail_off + t * 16, 16)]
                locbuf[pl.ds(t * 16, 16)] = lv
            return cntv - nb * _G

        cntv = lax.fori_loop(0, _SCHUNKS, scan_chunk,
                             jnp.zeros((16,), jnp.int32))

        cnt = cntv[0]
        for t in range(2):
            plsc.store_scatter(eidbuf, [cnt + t * 16 + iota16], dummy_eid)
            plsc.store_scatter(locbuf, [cnt + t * 16 + iota16], dummy_loc)
        gstart(0, rstage0, sem0)
        gdrain(rstage0, sem0)
        accum(0, rstage0)

        pltpu.sync_copy(slabf.at[pl.ds(0, _RANGE * 512)],
                        outf_hbm.at[pl.ds(lo * 512, _RANGE * 512)])


def _scatter_rows(rows, i_idx, zeros):
    mesh = plsc.VectorSubcoreMesh(core_axis_name="c", subcore_axis_name="s")
    return pl.kernel(
        _kc_body,
        out_type=jax.ShapeDtypeStruct((_NPAD * 512,), jnp.float32),
        mesh=mesh,
        compiler_params=pltpu.CompilerParams(needs_layout_passes=False),
        scratch_types=[
            pltpu.VMEM((_SCCH,), jnp.int32),
            pltpu.VMEM((_LCAP,), jnp.int32),
            pltpu.VMEM((_LCAP,), jnp.int32),
            pltpu.VMEM((_G, 512), jnp.float32),
            pltpu.VMEM((_G, 512), jnp.float32),
            pltpu.VMEM((_SLAB * 512,), jnp.float32),
            pltpu.SemaphoreType.DMA,
            pltpu.SemaphoreType.DMA,
        ],
    )(rows, i_idx, zeros)


def kernel(x, vec, edge_index, edge_rbf, edge_udiff, W1, b1, W2, b2, We, be):
    j = edge_index[0].astype(jnp.int32)
    i = edge_index[1].astype(jnp.int32)
    vec_flat = vec.reshape(N_NODES, 384)
    table = _build_table(x, vec_flat, W1, b1.reshape(1, -1), W2,
                         b2.reshape(1, -1))
    ej = _gather_edges(table, j)
    u0 = edge_udiff[:, 0:1]
    u1 = edge_udiff[:, 1:2]
    u2 = edge_udiff[:, 2:3]
    rows = _edge_dense(edge_rbf, ej, u0, u1, u2, We, be.reshape(1, -1))
    zeros = jnp.zeros((_SLAB * 512,), jnp.float32)
    out = _scatter_rows(rows, i, zeros).reshape(_NPAD, 512)[:N_NODES]
    d_vec = out[:, 0:384].reshape(N_NODES, 3, H)
    d_x = out[:, 384:512]
    return (d_x, d_vec)
